# Initial kernel scaffold; baseline (speedup 1.0000x reference)
import jax
import jax.numpy as jnp
from jax.experimental import pallas as pl


def kernel(x, edge_index):
    raise NotImplementedError("write your pallas kernel here")


# speedup vs baseline: 8.6034x; 8.6034x over previous
import functools

import jax
import jax.numpy as jnp
from jax import lax
from jax.experimental import pallas as pl
from jax.experimental.pallas import tpu as pltpu
from jax.experimental.pallas import tpu_sc as plsc

_N = 10000
_D = 128
_E = 320000
_NC = 2
_NS = 16
_NW = _NC * _NS
_B = 128
_K = -(-_E // (_NW * _B))
_EP = _NW * _B * _K
_NPAD = 10240
_RPT = _NPAD // _NS


def _sc_gather_scatter(x, src3, dst3):
    mesh = plsc.VectorSubcoreMesh(core_axis_name="c", subcore_axis_name="s")

    @functools.partial(
        pl.kernel,
        mesh=mesh,
        out_type=jax.ShapeDtypeStruct((_NC, _N, _D), jnp.float32),
        scratch_types=[
            pltpu.VMEM((_K, _B), jnp.int32),
            pltpu.VMEM((_K, _B), jnp.int32),
            pltpu.VMEM((_B, _D), jnp.float32),
            pltpu.VMEM_SHARED((_NPAD, _D), jnp.float32),
            pltpu.SemaphoreType.DMA,
        ],
    )
    def k(x_hbm, src_hbm, dst_hbm, out_hbm, src_v, dst_v, rows_v, acc_sh, sem):
        cid = lax.axis_index("c")
        sid = lax.axis_index("s")
        wid = sid * _NC + cid

        pltpu.sync_copy(src_hbm.at[wid], src_v)
        pltpu.sync_copy(dst_hbm.at[wid], dst_v)

        def zbody(i, carry):
            r = i // (_D // 16)
            col = (i % (_D // 16)) * 16
            rows_v[r, pl.ds(col, 16)] = jnp.zeros((16,), jnp.float32)
            return carry

        lax.fori_loop(0, _B * (_D // 16), zbody, 0)
        for t in range(_RPT // _B):
            pltpu.sync_copy(rows_v, acc_sh.at[pl.ds(sid * _RPT + t * _B, _B)])
        plsc.subcore_barrier()

        def body(w, carry):
            pltpu.async_copy(x_hbm.at[src_v.at[w]], rows_v, sem).wait()
            pltpu.sync_copy(rows_v, acc_sh.at[dst_v.at[w]], add=True)
            return carry

        lax.fori_loop(0, _K, body, 0)
        plsc.subcore_barrier()

        row0 = sid * _RPT

        @pl.when(sid < _NS - 1)
        def _():
            pltpu.sync_copy(acc_sh.at[pl.ds(row0, _RPT)],
                            out_hbm.at[cid, pl.ds(row0, _RPT)])

        @pl.when(sid == _NS - 1)
        def _():
            last = (_NS - 1) * _RPT
            pltpu.sync_copy(acc_sh.at[pl.ds(last, _N - last)],
                            out_hbm.at[cid, pl.ds(last, _N - last)])

    return k(x, src3, dst3)


def _combine_partials(partials):
    def add_body(a_ref, b_ref, o_ref):
        o_ref[...] = a_ref[...] + b_ref[...]

    blk = _N // 10
    return pl.pallas_call(
        add_body,
        out_shape=jax.ShapeDtypeStruct((_N, _D), jnp.float32),
        grid=(10,),
        in_specs=[
            pl.BlockSpec((blk, _D), lambda i: (i, 0)),
            pl.BlockSpec((blk, _D), lambda i: (i, 0)),
        ],
        out_specs=pl.BlockSpec((blk, _D), lambda i: (i, 0)),
    )(partials[0], partials[1])


def kernel(x, edge_index):
    pad = _EP - _E
    pad_ids = jnp.arange(pad, dtype=jnp.int32)
    src_pad = (pad_ids * 37) % _N
    dst_pad = _N + pad_ids % (_NPAD - _N)
    src3 = jnp.concatenate([edge_index[0], src_pad]).reshape(_NW, _K, _B)
    dst3 = jnp.concatenate([edge_index[1], dst_pad]).reshape(_NW, _K, _B)
    partials = _sc_gather_scatter(x, src3, dst3)
    return _combine_partials(partials)


# speedup vs baseline: 13.5853x; 1.5791x over previous
import functools

import jax
import jax.numpy as jnp
from jax import lax
from jax.experimental import pallas as pl
from jax.experimental.pallas import tpu as pltpu
from jax.experimental.pallas import tpu_sc as plsc

_N = 10000
_D = 128
_H = _D // 2
_E = 320000
_NC = 2
_NS = 16
_B = 128
_W = 156
_EPT = _W * _B
_REM = _E - _NS * _EPT
_NPAD = 10240
_RPT = _NPAD // _NS


def _sc_gather_scatter(xh, edges):
    mesh = plsc.VectorSubcoreMesh(core_axis_name="c", subcore_axis_name="s")

    @functools.partial(
        pl.kernel,
        mesh=mesh,
        compiler_params=pltpu.CompilerParams(use_tc_tiling_on_sc=False),
        out_type=jax.ShapeDtypeStruct((_N, _D), jnp.float32),
        scratch_types=[
            pltpu.VMEM((2, _B), jnp.int32),
            pltpu.VMEM((2, _B), jnp.int32),
            pltpu.VMEM((2, _B), jnp.int32),
            pltpu.VMEM((2, _B), jnp.int32),
            pltpu.VMEM((_B, _H), jnp.float32),
            pltpu.VMEM((_B, _H), jnp.float32),
            pltpu.VMEM_SHARED((_NPAD, _H), jnp.float32),
            pltpu.SemaphoreType.DMA,
            pltpu.SemaphoreType.DMA,
            pltpu.SemaphoreType.DMA,
            pltpu.SemaphoreType.DMA,
            pltpu.SemaphoreType.DMA,
            pltpu.SemaphoreType.DMA,
        ],
    )
    def k(x_hbm, e_hbm, out_hbm, sraw_v, draw_v, sidx_v, duse_v,
          rows0_v, rows1_v, acc_sh,
          isem0, isem1, jsem0, jsem1, gsem0, gsem1):
        cid = lax.axis_index("c")
        sid = lax.axis_index("s")
        base0 = sid * _EPT

        rows = (rows0_v, rows1_v)
        isems = (isem0, isem1)
        jsems = (jsem0, jsem1)
        gsems = (gsem0, gsem1)

        def idx_dma(w, b):
            s = pltpu.make_async_copy(
                e_hbm.at[0, pl.ds(base0 + w * _B, _B)], sraw_v.at[b], isems[b])
            d = pltpu.make_async_copy(
                e_hbm.at[1, pl.ds(base0 + w * _B, _B)], draw_v.at[b], jsems[b])
            return s, d

        def gather(w, b):
            return pltpu.make_async_copy(
                x_hbm.at[sidx_v.at[b]], rows[b], gsems[b])

        def prep_indices(b):
            cvec = jnp.full((16,), cid, jnp.int32)
            for j in range(_B // 16):
                sl = pl.ds(j * 16, 16)
                sidx_v[b, sl] = sraw_v[b, sl] * 2 + cvec
                duse_v[b, sl] = draw_v[b, sl]

        def zbody(i, carry):
            r = i // (_H // 16)
            col = (i % (_H // 16)) * 16
            rows0_v[r, pl.ds(col, 16)] = jnp.zeros((16,), jnp.float32)
            return carry

        lax.fori_loop(0, _B * (_H // 16), zbody, 0)
        for t in range(_RPT // _B):
            pltpu.sync_copy(rows0_v,
                            acc_sh.at[pl.ds(sid * _RPT + t * _B, _B)])
        plsc.subcore_barrier()

        for b in (0, 1):
            s, d = idx_dma(b, b)
            s.start(); d.start()
        for b in (0, 1):
            s, d = idx_dma(b, b)
            s.wait(); d.wait()
            prep_indices(b)
            gather(b, b).start()
        for b in (0, 1):
            s, d = idx_dma(2 + b, b)
            s.start(); d.start()

        def slot(w, b):
            gather(w, b).wait()
            pltpu.sync_copy(rows[b], acc_sh.at[duse_v.at[b]], add=True)

            @pl.when(w + 2 < _W)
            def _():
                s, d = idx_dma(w + 2, b)
                s.wait(); d.wait()
                prep_indices(b)
                gather(w + 2, b).start()

            @pl.when(w + 4 < _W)
            def _():
                s, d = idx_dma(w + 4, b)
                s.start(); d.start()

        def body(p, carry):
            slot(2 * p, 0)
            slot(2 * p + 1, 1)
            return carry

        lax.fori_loop(0, _W // 2, body, 0)

        @pl.when(sid < _REM // _B)
        def _():
            rbase = _NS * _EPT + sid * _B
            pltpu.sync_copy(e_hbm.at[0, pl.ds(rbase, _B)], sraw_v.at[0])
            pltpu.sync_copy(e_hbm.at[1, pl.ds(rbase, _B)], draw_v.at[0])
            prep_indices(0)
            pltpu.make_async_copy(x_hbm.at[sidx_v.at[0]], rows0_v,
                                  gsem0).start()
            pltpu.make_async_copy(x_hbm.at[sidx_v.at[0]], rows0_v,
                                  gsem0).wait()
            pltpu.sync_copy(rows0_v, acc_sh.at[duse_v.at[0]], add=True)

        plsc.subcore_barrier()

        row0 = sid * _RPT

        @pl.when(sid < _NS - 1)
        def _():
            pltpu.sync_copy(acc_sh.at[pl.ds(row0, _RPT)],
                            out_hbm.at[pl.ds(row0, _RPT),
                                       pl.ds(cid * _H, _H)])

        @pl.when(sid == _NS - 1)
        def _():
            last = (_NS - 1) * _RPT
            pltpu.sync_copy(acc_sh.at[pl.ds(last, _N - last)],
                            out_hbm.at[pl.ds(last, _N - last),
                                       pl.ds(cid * _H, _H)])

    return k(xh, edges)


def kernel(x, edge_index):
    xh = x.reshape(2 * _N, _H)
    return _sc_gather_scatter(xh, edge_index)


# speedup vs baseline: 14.0127x; 1.0315x over previous
import functools

import jax
import jax.numpy as jnp
from jax import lax
from jax.experimental import pallas as pl
from jax.experimental.pallas import tpu as pltpu
from jax.experimental.pallas import tpu_sc as plsc

_N = 10000
_D = 128
_H = _D // 2
_E = 320000
_NC = 2
_NS = 16
_B = 128
_W = 156
_EPT = _W * _B
_REM = _E - _NS * _EPT
_NPAD = 10240
_RPT = _NPAD // _NS
_R = 6
_IR = 8


def _sc_gather_scatter(xh, edges):
    mesh = plsc.VectorSubcoreMesh(core_axis_name="c", subcore_axis_name="s")

    @functools.partial(
        pl.kernel,
        mesh=mesh,
        compiler_params=pltpu.CompilerParams(use_tc_tiling_on_sc=False),
        out_type=jax.ShapeDtypeStruct((_N, _D), jnp.float32),
        scratch_types=(
            [pltpu.VMEM((_IR, 2, _B), jnp.int32)]
            + [pltpu.VMEM((_IR, _B), jnp.int32)]
            + [pltpu.VMEM((_IR, _B), jnp.int32)]
            + [pltpu.VMEM((_B, _H), jnp.float32) for _ in range(_R)]
            + [pltpu.VMEM_SHARED((_NPAD, _H), jnp.float32)]
            + [pltpu.SemaphoreType.DMA] * (3 * _R)
        ),
    )
    def k(x_hbm, e_hbm, out_hbm, sdraw_v, sidx_v, duse_v,
          r0, r1, r2, r3, r4, r5, acc_sh, *sems):
        isems = sems[0:_R]
        gsems = sems[_R:2 * _R]
        ssems = sems[2 * _R:3 * _R]
        rows = (r0, r1, r2, r3, r4, r5)
        cid = lax.axis_index("c")
        sid = lax.axis_index("s")
        base0 = sid * _EPT

        def idx_dma(w, js):
            return pltpu.make_async_copy(
                e_hbm.at[:, pl.ds(base0 + w * _B, _B)],
                sdraw_v.at[w % _IR], isems[js])

        def gather(w, jr):
            return pltpu.make_async_copy(
                x_hbm.at[sidx_v.at[w % _IR]], rows[jr], gsems[jr])

        def scat_wait(jr):
            return pltpu.make_async_copy(
                rows[jr], acc_sh.at[duse_v.at[0]], ssems[jr])

        def prep_indices(w):
            r8 = w % _IR
            cvec = jnp.full((16,), cid, jnp.int32)
            for j in range(_B // 16):
                sl = pl.ds(j * 16, 16)
                sidx_v[r8, sl] = sdraw_v[r8, 0, sl] * 2 + cvec
                duse_v[r8, sl] = sdraw_v[r8, 1, sl]

        def zbody(i, carry):
            r = i // (_H // 16)
            col = (i % (_H // 16)) * 16
            r0[r, pl.ds(col, 16)] = jnp.zeros((16,), jnp.float32)
            return carry

        lax.fori_loop(0, _B * (_H // 16), zbody, 0)
        for t in range(_RPT // _B):
            pltpu.sync_copy(r0, acc_sh.at[pl.ds(sid * _RPT + t * _B, _B)])
        plsc.subcore_barrier()

        for v in range(4):
            idx_dma(v, v % _R).start()
        for v in range(2):
            idx_dma(v, v % _R).wait()
            prep_indices(v)
            gather(v, v % _R).start()

        def visit(v, j):
            gather(v, j).wait()
            pltpu.async_copy(rows[j], acc_sh.at[duse_v.at[v % _IR]],
                             ssems[j], add=True)

            @pl.when(v + 2 < _W)
            def _():
                j2 = (j + 2) % _R
                idx_dma(v + 2, j2).wait()
                prep_indices(v + 2)

                @pl.when(v >= 4)
                def _():
                    scat_wait(j2).wait()

                gather(v + 2, j2).start()

            @pl.when(v + 4 < _W)
            def _():
                idx_dma(v + 4, (j + 4) % _R).start()

        def body(q, carry):
            for j in range(_R):
                visit(q * _R + j, j)
            return carry

        lax.fori_loop(0, _W // _R, body, 0)

        for j in range(_R):
            scat_wait(j).wait()

        @pl.when(sid < _REM // _B)
        def _():
            rbase = _NS * _EPT + sid * _B
            pltpu.sync_copy(e_hbm.at[:, pl.ds(rbase, _B)], sdraw_v.at[0])
            prep_indices(0)
            gather(0, 0).start()
            gather(0, 0).wait()
            pltpu.sync_copy(r0, acc_sh.at[duse_v.at[0]], add=True)

        plsc.subcore_barrier()

        row0 = sid * _RPT

        @pl.when(sid < _NS - 1)
        def _():
            pltpu.sync_copy(acc_sh.at[pl.ds(row0, _RPT)],
                            out_hbm.at[pl.ds(row0, _RPT),
                                       pl.ds(cid * _H, _H)])

        @pl.when(sid == _NS - 1)
        def _():
            last = (_NS - 1) * _RPT
            pltpu.sync_copy(acc_sh.at[pl.ds(last, _N - last)],
                            out_hbm.at[pl.ds(last, _N - last),
                                       pl.ds(cid * _H, _H)])

    return k(xh, edges)


def kernel(x, edge_index):
    xh = x.reshape(2 * _N, _H)
    return _sc_gather_scatter(xh, edge_index)


# speedup vs baseline: 17.2085x; 1.2281x over previous
import functools

import jax
import jax.numpy as jnp
from jax import lax
from jax.experimental import pallas as pl
from jax.experimental.pallas import tpu as pltpu
from jax.experimental.pallas import tpu_sc as plsc

_N = 10000
_D = 128
_H = _D // 2
_E = 320000
_NC = 2
_NS = 16
_B = 128
_W = 156
_EPT = _W * _B
_REM = _E - _NS * _EPT
_NPAD = 10240
_RPT = _NPAD // _NS
_R = 6
_IR = 8


def _sc_gather_scatter(xh, edges):
    mesh = plsc.VectorSubcoreMesh(core_axis_name="c", subcore_axis_name="s")

    @functools.partial(
        pl.kernel,
        mesh=mesh,
        compiler_params=pltpu.CompilerParams(use_tc_tiling_on_sc=False),
        out_type=jax.ShapeDtypeStruct((_N, _D), jnp.float32),
        scratch_types=(
            [pltpu.VMEM((_IR, 2, _B), jnp.int32)]
            + [pltpu.VMEM((_IR, _B), jnp.int32)]
            + [pltpu.VMEM((_IR, _B), jnp.int32)]
            + [pltpu.VMEM((_B, _H), jnp.float32) for _ in range(_R)]
            + [pltpu.VMEM_SHARED((_NPAD, _H), jnp.float32)]
            + [pltpu.SemaphoreType.DMA] * (3 * _R)
        ),
    )
    def k(x_hbm, e_hbm, out_hbm, sdraw_v, sidx_v, duse_v,
          r0, r1, r2, r3, r4, r5, acc_sh, *sems):
        isems = sems[0:_R]
        gsems = sems[_R:2 * _R]
        ssems = sems[2 * _R:3 * _R]
        rows = (r0, r1, r2, r3, r4, r5)
        cid = lax.axis_index("c")
        sid = lax.axis_index("s")
        base0 = sid * _EPT

        def idx_dma(w, js):
            return pltpu.make_async_copy(
                e_hbm.at[:, pl.ds(base0 + w * _B, _B)],
                sdraw_v.at[w % _IR], isems[js])

        def gather(w, jr):
            return pltpu.make_async_copy(
                x_hbm.at[sidx_v.at[w % _IR]], rows[jr], gsems[jr])

        def scat_wait(jr):
            return pltpu.make_async_copy(
                rows[jr], acc_sh.at[duse_v.at[0]], ssems[jr])

        def prep_indices(w):
            r8 = w % _IR
            cvec = jnp.full((16,), cid, jnp.int32)
            for j in range(_B // 16):
                sl = pl.ds(j * 16, 16)
                sidx_v[r8, sl] = sdraw_v[r8, 0, sl] * 2 + cvec
                duse_v[r8, sl] = sdraw_v[r8, 1, sl]

        def zbody(i, carry):
            r = i // (_H // 16)
            col = (i % (_H // 16)) * 16
            r0[r, pl.ds(col, 16)] = jnp.zeros((16,), jnp.float32)
            return carry

        lax.fori_loop(0, _B * (_H // 16), zbody, 0)
        for t in range(_RPT // _B):
            pltpu.sync_copy(r0, acc_sh.at[pl.ds(sid * _RPT + t * _B, _B)])
        plsc.subcore_barrier()

        for v in range(6):
            idx_dma(v, v % _R).start()
        for v in range(4):
            idx_dma(v, v % _R).wait()
            prep_indices(v)
            gather(v, v % _R).start()

        def visit(v, j):
            gather(v, j).wait()
            pltpu.async_copy(rows[j], acc_sh.at[duse_v.at[v % _IR]],
                             ssems[j], add=True)

            @pl.when(v + 4 < _W)
            def _():
                j4 = (j + 4) % _R
                idx_dma(v + 4, j4).wait()
                prep_indices(v + 4)

                @pl.when(v >= 2)
                def _():
                    scat_wait(j4).wait()

                gather(v + 4, j4).start()

            @pl.when(v + 6 < _W)
            def _():
                idx_dma(v + 6, j).start()

        def body(q, carry):
            for j in range(_R):
                visit(q * _R + j, j)
            return carry

        lax.fori_loop(0, _W // _R, body, 0)

        for j in range(_R):
            scat_wait(j).wait()


        @pl.when(sid < _REM // _B)
        def _():
            rbase = _NS * _EPT + sid * _B
            pltpu.sync_copy(e_hbm.at[:, pl.ds(rbase, _B)], sdraw_v.at[0])
            prep_indices(0)
            gather(0, 0).start()
            gather(0, 0).wait()
            pltpu.sync_copy(r0, acc_sh.at[duse_v.at[0]], add=True)

        plsc.subcore_barrier()

        row0 = sid * _RPT

        @pl.when(sid < _NS - 1)
        def _():
            pltpu.sync_copy(acc_sh.at[pl.ds(row0, _RPT)],
                            out_hbm.at[pl.ds(row0, _RPT),
                                       pl.ds(cid * _H, _H)])

        @pl.when(sid == _NS - 1)
        def _():
            last = (_NS - 1) * _RPT
            pltpu.sync_copy(acc_sh.at[pl.ds(last, _N - last)],
                            out_hbm.at[pl.ds(last, _N - last),
                                       pl.ds(cid * _H, _H)])

    return k(xh, edges)


def kernel(x, edge_index):
    xh = x.reshape(2 * _N, _H)
    return _sc_gather_scatter(xh, edge_index)
